# Initial kernel scaffold; baseline (speedup 1.0000x reference)
import math

import jax
import jax.numpy as jnp
from jax import lax
from jax.experimental import pallas as pl
from jax.experimental.pallas import tpu as pltpu

BS = 128
CRITICAL_T = 0.5
NEGLIGIBLE_T = 0.1
NEG = -1e9


def _fmap(x):
    return jnp.where(x > 0, x + 1.0, jnp.exp(x))


def _stats_kernel(q_ref, k_ref, v_ref, kvp_ref, ksp_ref, qrep_ref, krep_ref):
    T, D = q_ref.shape[1], q_ref.shape[2]
    n = T // BS
    kv = jnp.zeros((D, D), jnp.float32)
    ks = jnp.zeros((1, D), jnp.float32)
    for i in range(n):
        qb = q_ref[0, i * BS:(i + 1) * BS, :]
        kb = k_ref[0, i * BS:(i + 1) * BS, :]
        vb = v_ref[0, i * BS:(i + 1) * BS, :]
        kf = _fmap(kb)
        kv = kv + lax.dot_general(kf, vb, (((0,), (0,)), ((), ())),
                                  preferred_element_type=jnp.float32)
        ks = ks + jnp.sum(kf, axis=0, keepdims=True)
        kvp_ref[0, i] = kv
        ksp_ref[0, i:i + 1, :] = ks
        qrep_ref[0, i:i + 1, :] = jnp.mean(qb, axis=0, keepdims=True)
        krep_ref[0, i:i + 1, :] = jnp.mean(kb, axis=0, keepdims=True)


def _mode_kernel(qrep_ref, krep_ref, mode_ref):
    H, n, D = qrep_ref.shape
    scale = 1.0 / math.sqrt(D)
    row = lax.broadcasted_iota(jnp.int32, (n, n), 0)
    col = lax.broadcasted_iota(jnp.int32, (n, n), 1)
    causal = col <= row
    masked = []
    for h in range(H):
        s = lax.dot_general(qrep_ref[h], krep_ref[h], (((1,), (1,)), ((), ())),
                            preferred_element_type=jnp.float32) * scale
        masked.append(jnp.where(causal, s, NEG))
    gmin = jnp.min(masked[0])
    for h in range(1, H):
        gmin = jnp.minimum(gmin, jnp.min(masked[h]))
    imps = []
    for h in range(H):
        s = masked[h]
        m = jnp.max(s, axis=1, keepdims=True)
        e = jnp.exp(s - m)
        sm = e / jnp.sum(e, axis=1, keepdims=True)
        mu = jnp.mean(sm, axis=1, keepdims=True)
        var = jnp.sum((sm - mu) ** 2, axis=1, keepdims=True) / (n - 1)
        imps.append(var * jnp.maximum(m - gmin, 1e-6))
    gmax_i = jnp.max(imps[0])
    gmin_i = jnp.min(imps[0])
    for h in range(1, H):
        gmax_i = jnp.maximum(gmax_i, jnp.max(imps[h]))
        gmin_i = jnp.minimum(gmin_i, jnp.min(imps[h]))
    denom = gmax_i - gmin_i + 1e-6
    lane = lax.broadcasted_iota(jnp.int32, (n, H), 1)
    modes = jnp.zeros((n, H), jnp.int32)
    for h in range(H):
        inorm = (imps[h] - gmin_i) / denom
        mcol = jnp.where(inorm >= CRITICAL_T, 2,
                         jnp.where(inorm <= NEGLIGIBLE_T, 0, 1)).astype(jnp.int32)
        modes = jnp.where(lane == h, mcol, modes)
    mode_ref[...] = modes


def _attn_kernel(mode_ref, q_ref, k_ref, v_ref, kvp_ref, ksp_ref, out_ref):
    h = pl.program_id(0)
    qi = pl.program_id(1)
    D = q_ref.shape[2]
    scale = 1.0 / math.sqrt(D)
    mode = mode_ref[qi, h]

    @pl.when(mode == 2)
    def _crit():
        qv = q_ref[0]

        def body(ki, acc):
            kb = k_ref[0, pl.ds(ki * BS, BS), :]
            vb = v_ref[0, pl.ds(ki * BS, BS), :]
            s = lax.dot_general(qv, kb, (((1,), (1,)), ((), ())),
                                preferred_element_type=jnp.float32) * scale
            r = lax.broadcasted_iota(jnp.int32, (BS, BS), 0)
            c = lax.broadcasted_iota(jnp.int32, (BS, BS), 1)
            s = jnp.where((r >= c) | (ki < qi), s, NEG)
            m = jnp.max(s, axis=1, keepdims=True)
            e = jnp.exp(s - m)
            p = e / jnp.sum(e, axis=1, keepdims=True)
            return acc + lax.dot_general(p, vb, (((1,), (0,)), ((), ())),
                                         preferred_element_type=jnp.float32)

        acc = lax.fori_loop(0, qi + 1, body, jnp.zeros((BS, D), jnp.float32))
        out_ref[0] = acc / (qi + 1).astype(jnp.float32)

    @pl.when(mode == 1)
    def _marg():
        qf = _fmap(q_ref[0])
        num = jnp.dot(qf, kvp_ref[0, 0], preferred_element_type=jnp.float32)
        ks = ksp_ref[0, pl.ds(qi, 1), :]
        den = lax.dot_general(qf, ks, (((1,), (1,)), ((), ())),
                              preferred_element_type=jnp.float32)
        den = den + 1e-6 * (qi + 1).astype(jnp.float32)
        out_ref[0] = num / jnp.maximum(den, 1e-6)

    @pl.when(mode == 0)
    def _neg():
        out_ref[0] = jnp.zeros((BS, D), jnp.float32)


@jax.jit
def kernel(q, k, v):
    B, T, H, D = q.shape
    n = T // BS
    f32 = jnp.float32
    qt = jnp.transpose(q[0], (1, 0, 2))
    kt = jnp.transpose(k[0], (1, 0, 2))
    vt = jnp.transpose(v[0], (1, 0, 2))

    kvp, ksp, qrep, krep = pl.pallas_call(
        _stats_kernel,
        grid=(H,),
        in_specs=[pl.BlockSpec((1, T, D), lambda h: (h, 0, 0))] * 3,
        out_specs=[
            pl.BlockSpec((1, n, D, D), lambda h: (h, 0, 0, 0)),
            pl.BlockSpec((1, n, D), lambda h: (h, 0, 0)),
            pl.BlockSpec((1, n, D), lambda h: (h, 0, 0)),
            pl.BlockSpec((1, n, D), lambda h: (h, 0, 0)),
        ],
        out_shape=[
            jax.ShapeDtypeStruct((H, n, D, D), f32),
            jax.ShapeDtypeStruct((H, n, D), f32),
            jax.ShapeDtypeStruct((H, n, D), f32),
            jax.ShapeDtypeStruct((H, n, D), f32),
        ],
    )(qt, kt, vt)

    mode = pl.pallas_call(
        _mode_kernel,
        out_shape=jax.ShapeDtypeStruct((n, H), jnp.int32),
    )(qrep, krep)

    out_t = pl.pallas_call(
        _attn_kernel,
        grid_spec=pltpu.PrefetchScalarGridSpec(
            num_scalar_prefetch=1,
            grid=(H, n),
            in_specs=[
                pl.BlockSpec((1, BS, D), lambda h, qi, *_: (h, qi, 0)),
                pl.BlockSpec((1, T, D), lambda h, qi, *_: (h, 0, 0)),
                pl.BlockSpec((1, T, D), lambda h, qi, *_: (h, 0, 0)),
                pl.BlockSpec((1, 1, D, D), lambda h, qi, *_: (h, qi, 0, 0)),
                pl.BlockSpec((1, n, D), lambda h, qi, *_: (h, 0, 0)),
            ],
            out_specs=pl.BlockSpec((1, BS, D), lambda h, qi, *_: (h, qi, 0)),
        ),
        out_shape=jax.ShapeDtypeStruct((H, T, D), f32),
        compiler_params=pltpu.CompilerParams(
            dimension_semantics=("arbitrary", "arbitrary")),
    )(mode, qt, kt, vt, kvp, ksp)

    return jnp.transpose(out_t, (1, 0, 2))[None]


# speedup vs baseline: 5.3377x; 5.3377x over previous
import math

import jax
import jax.numpy as jnp
from jax import lax
from jax.experimental import pallas as pl
from jax.experimental.pallas import tpu as pltpu

BS = 128
CRITICAL_T = 0.5
NEGLIGIBLE_T = 0.1
NEG = -1e9


def _fmap(x):
    return jnp.where(x > 0, x + 1.0, jnp.exp(x))


def _stats_kernel(q_ref, k_ref, v_ref, kvp_ref, ksp_ref, qrep_ref, krep_ref):
    T, D = q_ref.shape[1], q_ref.shape[2]
    n = T // BS
    kv = jnp.zeros((D, D), jnp.float32)
    ks = jnp.zeros((1, D), jnp.float32)
    for i in range(n):
        qb = q_ref[0, i * BS:(i + 1) * BS, :]
        kb = k_ref[0, i * BS:(i + 1) * BS, :]
        vb = v_ref[0, i * BS:(i + 1) * BS, :]
        kf = _fmap(kb)
        kv = kv + lax.dot_general(kf, vb, (((0,), (0,)), ((), ())),
                                  preferred_element_type=jnp.float32)
        ks = ks + jnp.sum(kf, axis=0, keepdims=True)
        kvp_ref[0, i] = kv
        ksp_ref[0, i:i + 1, :] = ks
        qrep_ref[0, i:i + 1, :] = jnp.mean(qb, axis=0, keepdims=True)
        krep_ref[0, i:i + 1, :] = jnp.mean(kb, axis=0, keepdims=True)


def _mode_kernel(qrep_ref, krep_ref, mode_ref):
    H, n, D = qrep_ref.shape
    scale = 1.0 / math.sqrt(D)
    row = lax.broadcasted_iota(jnp.int32, (n, n), 0)
    col = lax.broadcasted_iota(jnp.int32, (n, n), 1)
    causal = col <= row
    masked = []
    for h in range(H):
        s = lax.dot_general(qrep_ref[h], krep_ref[h], (((1,), (1,)), ((), ())),
                            preferred_element_type=jnp.float32) * scale
        masked.append(jnp.where(causal, s, NEG))
    gmin = jnp.min(masked[0])
    for h in range(1, H):
        gmin = jnp.minimum(gmin, jnp.min(masked[h]))
    imps = []
    for h in range(H):
        s = masked[h]
        m = jnp.max(s, axis=1, keepdims=True)
        e = jnp.exp(s - m)
        sm = e / jnp.sum(e, axis=1, keepdims=True)
        mu = jnp.mean(sm, axis=1, keepdims=True)
        var = jnp.sum((sm - mu) ** 2, axis=1, keepdims=True) / (n - 1)
        imps.append(var * jnp.maximum(m - gmin, 1e-6))
    gmax_i = jnp.max(imps[0])
    gmin_i = jnp.min(imps[0])
    for h in range(1, H):
        gmax_i = jnp.maximum(gmax_i, jnp.max(imps[h]))
        gmin_i = jnp.minimum(gmin_i, jnp.min(imps[h]))
    denom = gmax_i - gmin_i + 1e-6
    lane = lax.broadcasted_iota(jnp.int32, (n, H), 1)
    modes = jnp.zeros((n, H), jnp.int32)
    for h in range(H):
        inorm = (imps[h] - gmin_i) / denom
        mcol = jnp.where(inorm >= CRITICAL_T, 2,
                         jnp.where(inorm <= NEGLIGIBLE_T, 0, 1)).astype(jnp.int32)
        modes = jnp.where(lane == h, mcol, modes)
    mode_ref[...] = modes


def _attn_kernel(mode_ref, q_ref, k_ref, v_ref, kvp_ref, ksp_ref, out_ref):
    h = pl.program_id(0)
    qi = pl.program_id(1)
    D = q_ref.shape[2]
    scale = 1.0 / math.sqrt(D)
    mode = mode_ref[qi, h]

    @pl.when(mode == 2)
    def _crit():
        qv = q_ref[0]

        def body(ki, acc):
            kb = k_ref[0, pl.ds(ki * BS, BS), :]
            vb = v_ref[0, pl.ds(ki * BS, BS), :]
            s = lax.dot_general(qv, kb, (((1,), (1,)), ((), ())),
                                preferred_element_type=jnp.float32) * scale
            r = lax.broadcasted_iota(jnp.int32, (BS, BS), 0)
            c = lax.broadcasted_iota(jnp.int32, (BS, BS), 1)
            s = jnp.where((r >= c) | (ki < qi), s, NEG)
            m = jnp.max(s, axis=1, keepdims=True)
            e = jnp.exp(s - m)
            p = e / jnp.sum(e, axis=1, keepdims=True)
            return acc + lax.dot_general(p, vb, (((1,), (0,)), ((), ())),
                                         preferred_element_type=jnp.float32)

        acc = lax.fori_loop(0, qi + 1, body, jnp.zeros((BS, D), jnp.float32))
        out_ref[0] = acc / (qi + 1).astype(jnp.float32)

    @pl.when(mode == 1)
    def _marg():
        qf = _fmap(q_ref[0])
        num = jnp.dot(qf, kvp_ref[0, 0], preferred_element_type=jnp.float32)
        ks = ksp_ref[0, pl.ds(qi, 1), :]
        den = jnp.sum(qf * ks, axis=1, keepdims=True)
        den = den + 1e-6 * (qi + 1).astype(jnp.float32)
        out_ref[0] = num / jnp.maximum(den, 1e-6)

    @pl.when(mode == 0)
    def _neg():
        out_ref[0] = jnp.zeros((BS, D), jnp.float32)


@jax.jit
def kernel(q, k, v):
    B, T, H, D = q.shape
    n = T // BS
    f32 = jnp.float32
    qt = jnp.transpose(q[0], (1, 0, 2))
    kt = jnp.transpose(k[0], (1, 0, 2))
    vt = jnp.transpose(v[0], (1, 0, 2))

    kvp, ksp, qrep, krep = pl.pallas_call(
        _stats_kernel,
        grid=(H,),
        in_specs=[pl.BlockSpec((1, T, D), lambda h: (h, 0, 0))] * 3,
        out_specs=[
            pl.BlockSpec((1, n, D, D), lambda h: (h, 0, 0, 0)),
            pl.BlockSpec((1, n, D), lambda h: (h, 0, 0)),
            pl.BlockSpec((1, n, D), lambda h: (h, 0, 0)),
            pl.BlockSpec((1, n, D), lambda h: (h, 0, 0)),
        ],
        out_shape=[
            jax.ShapeDtypeStruct((H, n, D, D), f32),
            jax.ShapeDtypeStruct((H, n, D), f32),
            jax.ShapeDtypeStruct((H, n, D), f32),
            jax.ShapeDtypeStruct((H, n, D), f32),
        ],
    )(qt, kt, vt)

    mode = pl.pallas_call(
        _mode_kernel,
        out_shape=jax.ShapeDtypeStruct((n, H), jnp.int32),
    )(qrep, krep)

    out_t = pl.pallas_call(
        _attn_kernel,
        grid_spec=pltpu.PrefetchScalarGridSpec(
            num_scalar_prefetch=1,
            grid=(H, n),
            in_specs=[
                pl.BlockSpec((1, BS, D), lambda h, qi, *_: (h, qi, 0)),
                pl.BlockSpec((1, T, D), lambda h, qi, *_: (h, 0, 0)),
                pl.BlockSpec((1, T, D), lambda h, qi, *_: (h, 0, 0)),
                pl.BlockSpec((1, 1, D, D), lambda h, qi, *_: (h, qi, 0, 0)),
                pl.BlockSpec((1, n, D), lambda h, qi, *_: (h, 0, 0)),
            ],
            out_specs=pl.BlockSpec((1, BS, D), lambda h, qi, *_: (h, qi, 0)),
        ),
        out_shape=jax.ShapeDtypeStruct((H, T, D), f32),
        compiler_params=pltpu.CompilerParams(
            dimension_semantics=("arbitrary", "arbitrary")),
    )(mode, qt, kt, vt, kvp, ksp)

    return jnp.transpose(out_t, (1, 0, 2))[None]


# speedup vs baseline: 6.2048x; 1.1624x over previous
import math

import jax
import jax.numpy as jnp
from jax import lax
from jax.experimental import pallas as pl
from jax.experimental.pallas import tpu as pltpu

BS = 128
CRITICAL_T = 0.5
NEGLIGIBLE_T = 0.1
NEG = -1e9


def _fmap(x):
    return jnp.where(x > 0, x + 1.0, jnp.exp(x))


def _route_kernel(q_ref, k_ref, mode_ref, needkv_ref, lastkv_ref, lastq_ref,
                  qrep_s, krep_s):
    T, D = q_ref.shape
    n = T // BS
    H = qrep_s.shape[0]
    h = pl.program_id(0)
    for i in range(n):
        qrep_s[h, i:i + 1, :] = jnp.mean(q_ref[i * BS:(i + 1) * BS, :], axis=0,
                                         keepdims=True)
        krep_s[h, i:i + 1, :] = jnp.mean(k_ref[i * BS:(i + 1) * BS, :], axis=0,
                                         keepdims=True)

    @pl.when(h == H - 1)
    def _finalize():
        scale = 1.0 / math.sqrt(D)
        row = lax.broadcasted_iota(jnp.int32, (n, n), 0)
        col = lax.broadcasted_iota(jnp.int32, (n, n), 1)
        causal = col <= row
        masked = []
        for h2 in range(H):
            s = lax.dot_general(qrep_s[h2], krep_s[h2],
                                (((1,), (1,)), ((), ())),
                                preferred_element_type=jnp.float32) * scale
            masked.append(jnp.where(causal, s, NEG))
        gmin = jnp.min(masked[0])
        for h2 in range(1, H):
            gmin = jnp.minimum(gmin, jnp.min(masked[h2]))
        imps = []
        for h2 in range(H):
            s = masked[h2]
            m = jnp.max(s, axis=1, keepdims=True)
            e = jnp.exp(s - m)
            sm = e / jnp.sum(e, axis=1, keepdims=True)
            mu = jnp.mean(sm, axis=1, keepdims=True)
            var = jnp.sum((sm - mu) ** 2, axis=1, keepdims=True) / (n - 1)
            imps.append(var * jnp.maximum(m - gmin, 1e-6))
        gmax_i = jnp.max(imps[0])
        gmin_i = jnp.min(imps[0])
        for h2 in range(1, H):
            gmax_i = jnp.maximum(gmax_i, jnp.max(imps[h2]))
            gmin_i = jnp.minimum(gmin_i, jnp.min(imps[h2]))
        denom = gmax_i - gmin_i + 1e-6
        lane = lax.broadcasted_iota(jnp.int32, (n, H), 1)
        modes = jnp.zeros((n, H), jnp.int32)
        for h2 in range(H):
            inorm = (imps[h2] - gmin_i) / denom
            mcol = jnp.where(inorm >= CRITICAL_T, 2,
                             jnp.where(inorm <= NEGLIGIBLE_T, 0,
                                       1)).astype(jnp.int32)
            modes = jnp.where(lane == h2, mcol, modes)
        marg = (modes == 1).astype(jnp.float32)
        upper = (col >= row).astype(jnp.float32)
        needkv = (lax.dot_general(upper, marg, (((1,), (0,)), ((), ())),
                                  preferred_element_type=jnp.float32)
                  > 0.5).astype(jnp.int32)
        rown = lax.broadcasted_iota(jnp.int32, (n, H), 0)
        use_kv = jnp.where((needkv == 1) | (modes == 2), rown, -1)
        use_q = jnp.where(modes != 0, rown, -1)
        lkv = jnp.zeros((n, H), jnp.int32)
        lq = jnp.zeros((n, H), jnp.int32)
        cur_kv = jnp.zeros((1, H), jnp.int32)
        cur_q = jnp.zeros((1, H), jnp.int32)
        for i in range(n):
            cur_kv = jnp.maximum(cur_kv, use_kv[i:i + 1, :])
            cur_q = jnp.maximum(cur_q, use_q[i:i + 1, :])
            lkv = jnp.where(rown == i, cur_kv, lkv)
            lq = jnp.where(rown == i, cur_q, lq)
        mode_ref[...] = modes
        needkv_ref[...] = needkv
        lastkv_ref[...] = lkv
        lastq_ref[...] = lq


def _attn_kernel(mode_ref, needkv_ref, lastkv_ref, lastq_ref,
                 q_ref, k_ref, v_ref, kany_ref, vany_ref, out_ref,
                 kv_acc, ks_acc, kbuf, vbuf, ksem, vsem):
    h = pl.program_id(0)
    qi = pl.program_id(1)
    D = q_ref.shape[1]
    scale = 1.0 / math.sqrt(D)
    mode = mode_ref[qi, h]

    @pl.when(qi == 0)
    def _reset():
        kv_acc[...] = jnp.zeros_like(kv_acc)
        ks_acc[...] = jnp.zeros_like(ks_acc)

    @pl.when(needkv_ref[qi, h] == 1)
    def _extend():
        kf = _fmap(k_ref[...])
        kv_acc[...] += lax.dot_general(kf, v_ref[...], (((0,), (0,)), ((), ())),
                                       preferred_element_type=jnp.float32)
        ks_acc[...] += jnp.sum(kf, axis=0, keepdims=True)

    @pl.when(mode == 2)
    def _crit():
        qv = q_ref[...]

        def block_attn(kb, vb, acc, diag):
            s = lax.dot_general(qv, kb, (((1,), (1,)), ((), ())),
                                preferred_element_type=jnp.float32) * scale
            if diag:
                r = lax.broadcasted_iota(jnp.int32, (BS, BS), 0)
                c = lax.broadcasted_iota(jnp.int32, (BS, BS), 1)
                s = jnp.where(r >= c, s, NEG)
            m = jnp.max(s, axis=1, keepdims=True)
            e = jnp.exp(s - m)
            p = e / jnp.sum(e, axis=1, keepdims=True)
            return acc + lax.dot_general(p, vb, (((1,), (0,)), ((), ())),
                                         preferred_element_type=jnp.float32)

        def body(ki, acc):
            ck = pltpu.make_async_copy(
                kany_ref.at[pl.ds(ki * BS, BS), pl.ds(h * D, D)], kbuf, ksem)
            cv = pltpu.make_async_copy(
                vany_ref.at[pl.ds(ki * BS, BS), pl.ds(h * D, D)], vbuf, vsem)
            ck.start()
            cv.start()
            ck.wait()
            cv.wait()
            return block_attn(kbuf[...], vbuf[...], acc, diag=False)

        acc = lax.fori_loop(0, qi, body, jnp.zeros((BS, D), jnp.float32))
        acc = block_attn(k_ref[...], v_ref[...], acc, diag=True)
        out_ref[...] = acc / (qi + 1).astype(jnp.float32)

    @pl.when(mode == 1)
    def _marg():
        qf = _fmap(q_ref[...])
        num = jnp.dot(qf, kv_acc[...], preferred_element_type=jnp.float32)
        den = jnp.sum(qf * ks_acc[...], axis=1, keepdims=True)
        den = den + 1e-6 * (qi + 1).astype(jnp.float32)
        out_ref[...] = num / jnp.maximum(den, 1e-6)

    @pl.when(mode == 0)
    def _neg():
        out_ref[...] = jnp.zeros((BS, D), jnp.float32)


@jax.jit
def kernel(q, k, v):
    B, T, H, D = q.shape
    n = T // BS
    f32 = jnp.float32
    i32 = jnp.int32
    q2 = q.reshape(T, H * D)
    k2 = k.reshape(T, H * D)
    v2 = v.reshape(T, H * D)

    mode, needkv, lastkv, lastq = pl.pallas_call(
        _route_kernel,
        grid=(H,),
        in_specs=[
            pl.BlockSpec((T, D), lambda h: (0, h)),
            pl.BlockSpec((T, D), lambda h: (0, h)),
        ],
        out_specs=[pl.BlockSpec((n, H), lambda h: (0, 0))] * 4,
        out_shape=[jax.ShapeDtypeStruct((n, H), i32)] * 4,
        scratch_shapes=[
            pltpu.VMEM((H, n, D), f32),
            pltpu.VMEM((H, n, D), f32),
        ],
    )(q2, k2)

    out2 = pl.pallas_call(
        _attn_kernel,
        grid_spec=pltpu.PrefetchScalarGridSpec(
            num_scalar_prefetch=4,
            grid=(H, n),
            in_specs=[
                pl.BlockSpec((BS, D), lambda h, qi, *s: (s[3][qi, h], h)),
                pl.BlockSpec((BS, D), lambda h, qi, *s: (s[2][qi, h], h)),
                pl.BlockSpec((BS, D), lambda h, qi, *s: (s[2][qi, h], h)),
                pl.BlockSpec(memory_space=pl.ANY),
                pl.BlockSpec(memory_space=pl.ANY),
            ],
            out_specs=pl.BlockSpec((BS, D), lambda h, qi, *s: (qi, h)),
            scratch_shapes=[
                pltpu.VMEM((D, D), f32),
                pltpu.VMEM((1, D), f32),
                pltpu.VMEM((BS, D), f32),
                pltpu.VMEM((BS, D), f32),
                pltpu.SemaphoreType.DMA,
                pltpu.SemaphoreType.DMA,
            ],
        ),
        out_shape=jax.ShapeDtypeStruct((T, H * D), f32),
        compiler_params=pltpu.CompilerParams(
            dimension_semantics=("arbitrary", "arbitrary")),
    )(mode, needkv, lastkv, lastq, q2, k2, v2, k2, v2)

    return out2.reshape(B, T, H, D)


# speedup vs baseline: 10.9648x; 1.7671x over previous
import math

import jax
import jax.numpy as jnp
from jax import lax
from jax.experimental import pallas as pl
from jax.experimental.pallas import tpu as pltpu

BS = 128
CRITICAL_T = 0.5
NEGLIGIBLE_T = 0.1
NEG = -1e9


def _fmap(x):
    return jnp.where(x > 0, x + 1.0, jnp.exp(x))


def _route_kernel(q_ref, k_ref, mode_ref, needkv_ref, lastuse_ref,
                  qrep_s, krep_s):
    n, H, D = qrep_s.shape
    i = pl.program_id(0)
    qrep_s[i] = jnp.mean(q_ref[0], axis=0)
    krep_s[i] = jnp.mean(k_ref[0], axis=0)

    @pl.when(i == n - 1)
    def _finalize():
        scale = 1.0 / math.sqrt(D)
        row = lax.broadcasted_iota(jnp.int32, (n, n), 0)
        col = lax.broadcasted_iota(jnp.int32, (n, n), 1)
        causal = col <= row
        masked = []
        for h2 in range(H):
            s = lax.dot_general(qrep_s[:, h2, :], krep_s[:, h2, :],
                                (((1,), (1,)), ((), ())),
                                preferred_element_type=jnp.float32) * scale
            masked.append(jnp.where(causal, s, NEG))
        gmin = jnp.min(masked[0])
        for h2 in range(1, H):
            gmin = jnp.minimum(gmin, jnp.min(masked[h2]))
        imps = []
        for h2 in range(H):
            s = masked[h2]
            m = jnp.max(s, axis=1, keepdims=True)
            e = jnp.exp(s - m)
            sm = e / jnp.sum(e, axis=1, keepdims=True)
            mu = jnp.mean(sm, axis=1, keepdims=True)
            var = jnp.sum((sm - mu) ** 2, axis=1, keepdims=True) / (n - 1)
            imps.append(var * jnp.maximum(m - gmin, 1e-6))
        gmax_i = jnp.max(imps[0])
        gmin_i = jnp.min(imps[0])
        for h2 in range(1, H):
            gmax_i = jnp.maximum(gmax_i, jnp.max(imps[h2]))
            gmin_i = jnp.minimum(gmin_i, jnp.min(imps[h2]))
        denom = gmax_i - gmin_i + 1e-6
        lane = lax.broadcasted_iota(jnp.int32, (n, H), 1)
        modes = jnp.zeros((n, H), jnp.int32)
        for h2 in range(H):
            inorm = (imps[h2] - gmin_i) / denom
            mcol = jnp.where(inorm >= CRITICAL_T, 2,
                             jnp.where(inorm <= NEGLIGIBLE_T, 0,
                                       1)).astype(jnp.int32)
            modes = jnp.where(lane == h2, mcol, modes)
        marg = (modes == 1).astype(jnp.float32)
        upper = (col >= row).astype(jnp.float32)
        needkv = (lax.dot_general(upper, marg, (((1,), (0,)), ((), ())),
                                  preferred_element_type=jnp.float32)
                  > 0.5).astype(jnp.int32)
        rown = lax.broadcasted_iota(jnp.int32, (n, H), 0)
        used = (modes != 0) | (needkv == 1)
        any_use = jnp.max(jnp.where(used, 1, 0), axis=1, keepdims=True)
        use_idx = jnp.where(any_use == 1,
                            lax.broadcasted_iota(jnp.int32, (n, 1), 0), -1)
        lu = jnp.zeros((n, H), jnp.int32)
        cur = jnp.zeros((1, 1), jnp.int32)
        for i2 in range(n):
            cur = jnp.maximum(cur, use_idx[i2:i2 + 1, :])
            lu = jnp.where(rown == i2, jnp.maximum(cur, 0), lu)
        mode_ref[...] = modes
        needkv_ref[...] = needkv
        lastuse_ref[...] = lu


def _attn_kernel(mode_ref, needkv_ref, lastuse_ref,
                 q_ref, k_ref, v_ref, kany_ref, vany_ref, out_ref,
                 kv_acc, ks_acc, kbuf, vbuf, ksem, vsem):
    qi = pl.program_id(0)
    h = pl.program_id(1)
    D = q_ref.shape[3]
    scale = 1.0 / math.sqrt(D)
    mode = mode_ref[qi, h]

    @pl.when(qi == 0)
    def _reset():
        kv_acc[h] = jnp.zeros_like(kv_acc[h])
        ks_acc[h] = jnp.zeros_like(ks_acc[h])

    @pl.when(needkv_ref[qi, h] == 1)
    def _extend():
        kf = _fmap(k_ref[0, :, h, :])
        kv_acc[h] += lax.dot_general(kf, v_ref[0, :, h, :],
                                     (((0,), (0,)), ((), ())),
                                     preferred_element_type=jnp.float32)
        ks_acc[h] += jnp.sum(kf, axis=0, keepdims=True)

    @pl.when(mode == 2)
    def _crit():
        qv = q_ref[0, :, h, :]

        def block_attn(kb, vb, acc, diag):
            s = lax.dot_general(qv, kb, (((1,), (1,)), ((), ())),
                                preferred_element_type=jnp.float32) * scale
            if diag:
                r = lax.broadcasted_iota(jnp.int32, (BS, BS), 0)
                c = lax.broadcasted_iota(jnp.int32, (BS, BS), 1)
                s = jnp.where(r >= c, s, NEG)
            m = jnp.max(s, axis=1, keepdims=True)
            e = jnp.exp(s - m)
            p = e / jnp.sum(e, axis=1, keepdims=True)
            return acc + lax.dot_general(p, vb, (((1,), (0,)), ((), ())),
                                         preferred_element_type=jnp.float32)

        def body(ki, acc):
            ck = pltpu.make_async_copy(
                kany_ref.at[0, pl.ds(ki * BS, BS), h, :], kbuf, ksem)
            cv = pltpu.make_async_copy(
                vany_ref.at[0, pl.ds(ki * BS, BS), h, :], vbuf, vsem)
            ck.start()
            cv.start()
            ck.wait()
            cv.wait()
            return block_attn(kbuf[...], vbuf[...], acc, diag=False)

        acc = lax.fori_loop(0, qi, body, jnp.zeros((BS, D), jnp.float32))
        acc = block_attn(k_ref[0, :, h, :], v_ref[0, :, h, :], acc, diag=True)
        out_ref[0, :, h, :] = acc / (qi + 1).astype(jnp.float32)

    @pl.when(mode == 1)
    def _marg():
        qf = _fmap(q_ref[0, :, h, :])
        num = jnp.dot(qf, kv_acc[h], preferred_element_type=jnp.float32)
        den = jnp.sum(qf * ks_acc[h], axis=1, keepdims=True)
        den = den + 1e-6 * (qi + 1).astype(jnp.float32)
        out_ref[0, :, h, :] = num / jnp.maximum(den, 1e-6)

    @pl.when(mode == 0)
    def _neg():
        out_ref[0, :, h, :] = jnp.zeros((BS, D), jnp.float32)


@jax.jit
def kernel(q, k, v):
    B, T, H, D = q.shape
    n = T // BS
    f32 = jnp.float32
    i32 = jnp.int32

    mode, needkv, lastuse = pl.pallas_call(
        _route_kernel,
        grid=(n,),
        in_specs=[
            pl.BlockSpec((1, BS, H, D), lambda i: (0, i, 0, 0)),
            pl.BlockSpec((1, BS, H, D), lambda i: (0, i, 0, 0)),
        ],
        out_specs=[pl.BlockSpec((n, H), lambda i: (0, 0))] * 3,
        out_shape=[jax.ShapeDtypeStruct((n, H), i32)] * 3,
        scratch_shapes=[
            pltpu.VMEM((n, H, D), f32),
            pltpu.VMEM((n, H, D), f32),
        ],
    )(q, k)

    out = pl.pallas_call(
        _attn_kernel,
        grid_spec=pltpu.PrefetchScalarGridSpec(
            num_scalar_prefetch=3,
            grid=(n, H),
            in_specs=[
                pl.BlockSpec((1, BS, H, D), lambda qi, h, *s: (0, s[2][qi, 0], 0, 0)),
                pl.BlockSpec((1, BS, H, D), lambda qi, h, *s: (0, s[2][qi, 0], 0, 0)),
                pl.BlockSpec((1, BS, H, D), lambda qi, h, *s: (0, s[2][qi, 0], 0, 0)),
                pl.BlockSpec(memory_space=pl.ANY),
                pl.BlockSpec(memory_space=pl.ANY),
            ],
            out_specs=pl.BlockSpec((1, BS, H, D), lambda qi, h, *s: (0, qi, 0, 0)),
            scratch_shapes=[
                pltpu.VMEM((H, D, D), f32),
                pltpu.VMEM((H, 1, D), f32),
                pltpu.VMEM((BS, D), f32),
                pltpu.VMEM((BS, D), f32),
                pltpu.SemaphoreType.DMA,
                pltpu.SemaphoreType.DMA,
            ],
        ),
        out_shape=jax.ShapeDtypeStruct((B, T, H, D), f32),
        compiler_params=pltpu.CompilerParams(
            dimension_semantics=("arbitrary", "arbitrary")),
    )(mode, needkv, lastuse, q, k, v, k, v)

    return out


# speedup vs baseline: 14.1304x; 1.2887x over previous
import math

import jax
import jax.numpy as jnp
from jax import lax
from jax.experimental import pallas as pl
from jax.experimental.pallas import tpu as pltpu

BS = 128
CRITICAL_T = 0.5
NEGLIGIBLE_T = 0.1
NEG = -1e9


def _fmap(x):
    return jnp.where(x > 0, x + 1.0, jnp.exp(x))


def _mega_kernel(q_hbm, k_hbm, v_hbm, out_hbm,
                 qcache, kcache, vbuf, obuf, kv_acc, ks_acc,
                 qrep_s, krep_s, tab_v, tab_s, slot_cur, slot_nxt,
                 qsem, ksem, vsem, osem, tsem, cvsem, cvbuf):
    n, _, H, D = qcache.shape
    gi = pl.program_id(0)
    h = pl.program_id(1)
    scale = 1.0 / math.sqrt(D)
    OFF_MODE, OFF_NKV, OFF_USE, OFF_NXT = 0, n, 2 * n, 3 * n

    def fetch_qk(i, sl):
        pltpu.make_async_copy(q_hbm.at[0, pl.ds(i * BS, BS), :, :],
                              qcache.at[sl], qsem.at[sl]).start()
        pltpu.make_async_copy(k_hbm.at[0, pl.ds(i * BS, BS), :, :],
                              kcache.at[sl], ksem.at[sl]).start()

    @pl.when((gi < n) & (h == 0))
    def _phase0():
        @pl.when(gi == 0)
        def _prime():
            for j in range(4):
                fetch_qk(j, j)

        pltpu.make_async_copy(q_hbm.at[0, pl.ds(gi * BS, BS), :, :],
                              qcache.at[gi], qsem.at[gi]).wait()
        pltpu.make_async_copy(k_hbm.at[0, pl.ds(gi * BS, BS), :, :],
                              kcache.at[gi], ksem.at[gi]).wait()
        qrep_s[gi] = jnp.mean(qcache[gi], axis=0)
        krep_s[gi] = jnp.mean(kcache[gi], axis=0)

        @pl.when(gi + 4 < n)
        def _ahead():
            fetch_qk(gi + 4, gi + 4)

        @pl.when(gi == n - 1)
        def _finalize():
            row = lax.broadcasted_iota(jnp.int32, (n, n), 0)
            col = lax.broadcasted_iota(jnp.int32, (n, n), 1)
            causal = col <= row
            masked = []
            for h2 in range(H):
                s = lax.dot_general(qrep_s[:, h2, :], krep_s[:, h2, :],
                                    (((1,), (1,)), ((), ())),
                                    preferred_element_type=jnp.float32) * scale
                masked.append(jnp.where(causal, s, NEG))
            gmin = jnp.min(masked[0])
            for h2 in range(1, H):
                gmin = jnp.minimum(gmin, jnp.min(masked[h2]))
            imps = []
            for h2 in range(H):
                s = masked[h2]
                m = jnp.max(s, axis=1, keepdims=True)
                e = jnp.exp(s - m)
                sm = e / jnp.sum(e, axis=1, keepdims=True)
                mu = jnp.mean(sm, axis=1, keepdims=True)
                var = jnp.sum((sm - mu) ** 2, axis=1, keepdims=True) / (n - 1)
                imps.append(var * jnp.maximum(m - gmin, 1e-6))
            gmax_i = jnp.max(imps[0])
            gmin_i = jnp.min(imps[0])
            for h2 in range(1, H):
                gmax_i = jnp.maximum(gmax_i, jnp.max(imps[h2]))
                gmin_i = jnp.minimum(gmin_i, jnp.min(imps[h2]))
            denom = gmax_i - gmin_i + 1e-6
            lane = lax.broadcasted_iota(jnp.int32, (n, H), 1)
            modes = jnp.zeros((n, H), jnp.int32)
            for h2 in range(H):
                inorm = (imps[h2] - gmin_i) / denom
                mcol = jnp.where(inorm >= CRITICAL_T, 2,
                                 jnp.where(inorm <= NEGLIGIBLE_T, 0,
                                           1)).astype(jnp.int32)
                modes = jnp.where(lane == h2, mcol, modes)
            marg = (modes == 1).astype(jnp.float32)
            upper = (col >= row).astype(jnp.float32)
            needkv = (lax.dot_general(upper, marg, (((1,), (0,)), ((), ())),
                                      preferred_element_type=jnp.float32)
                      > 0.5).astype(jnp.int32)
            used = (modes != 0) | (needkv == 1)
            anyuse = jnp.max(jnp.where(used, 1, 0), axis=1, keepdims=True)
            anyuse = jnp.broadcast_to(anyuse, (n, H)).astype(jnp.int32)
            nxt_rows = []
            cur = jnp.full((1, H), n, jnp.int32)
            for i2 in range(n - 1, -1, -1):
                cur = jnp.where(anyuse[i2:i2 + 1, :] == 1, i2, cur)
                nxt_rows.append(cur)
            nxt_rows.reverse()
            tab = jnp.concatenate(
                [modes, needkv, anyuse, jnp.concatenate(nxt_rows, axis=0)],
                axis=0)
            tab_v[...] = tab
            cp = pltpu.make_async_copy(tab_v, tab_s, tsem)
            cp.start()
            cp.wait()
            f0 = tab_s[OFF_NXT + 0, 0]
            slot_nxt[0, 0] = 0

            @pl.when(f0 < n)
            def _pf():
                pltpu.make_async_copy(v_hbm.at[0, pl.ds(f0 * BS, BS), :, :],
                                      vbuf.at[0], vsem.at[0]).start()

    qi = gi - n
    in_p1 = gi >= n

    @pl.when(in_p1)
    def _phase1():
        mode = tab_s[OFF_MODE + qi, h]
        nkv = tab_s[OFF_NKV + qi, h]
        au = tab_s[OFF_USE + qi, 0]
        os_ = lax.rem(qi, 2)

        @pl.when(qi == 0)
        def _reset():
            kv_acc[h] = jnp.zeros((D, D), jnp.float32)
            ks_acc[h] = jnp.zeros((1, D), jnp.float32)

        @pl.when(h == 0)
        def _head0():
            @pl.when(qi >= 2)
            def _drain():
                pltpu.make_async_copy(
                    obuf.at[os_],
                    out_hbm.at[0, pl.ds((qi - 2) * BS, BS), :, :],
                    osem.at[os_]).wait()

            @pl.when(au == 1)
            def _live():
                s = slot_nxt[0, 0]
                slot_cur[0, 0] = s
                slot_nxt[0, 0] = 1 - s
                pltpu.make_async_copy(
                    v_hbm.at[0, pl.ds(qi * BS, BS), :, :],
                    vbuf.at[s], vsem.at[s]).wait()

                @pl.when(qi < n - 1)
                def _pf_outer():
                    nx = tab_s[OFF_NXT + qi + 1, 0]

                    @pl.when(nx < n)
                    def _pf():
                        pltpu.make_async_copy(
                            v_hbm.at[0, pl.ds(nx * BS, BS), :, :],
                            vbuf.at[1 - s], vsem.at[1 - s]).start()

        vs = slot_cur[0, 0]

        @pl.when(nkv == 1)
        def _extend():
            kf = _fmap(kcache[qi, :, h, :])
            kv_acc[h] += lax.dot_general(kf, vbuf[vs, :, h, :],
                                         (((0,), (0,)), ((), ())),
                                         preferred_element_type=jnp.float32)
            ks_acc[h] += jnp.sum(kf, axis=0, keepdims=True)

        @pl.when(mode == 2)
        def _crit():
            qv = qcache[qi, :, h, :]

            def block_attn(kb, vb, acc, diag):
                s = lax.dot_general(qv, kb, (((1,), (1,)), ((), ())),
                                    preferred_element_type=jnp.float32) * scale
                if diag:
                    r = lax.broadcasted_iota(jnp.int32, (BS, BS), 0)
                    c = lax.broadcasted_iota(jnp.int32, (BS, BS), 1)
                    s = jnp.where(r >= c, s, NEG)
                m = jnp.max(s, axis=1, keepdims=True)
                e = jnp.exp(s - m)
                p = e / jnp.sum(e, axis=1, keepdims=True)
                return acc + lax.dot_general(p, vb, (((1,), (0,)), ((), ())),
                                             preferred_element_type=jnp.float32)

            def body(ki, acc):
                cv = pltpu.make_async_copy(
                    v_hbm.at[0, pl.ds(ki * BS, BS), h, :], cvbuf, cvsem)
                cv.start()
                cv.wait()
                return block_attn(kcache[ki, :, h, :], cvbuf[...], acc,
                                  diag=False)

            acc = lax.fori_loop(0, qi, body, jnp.zeros((BS, D), jnp.float32))
            acc = block_attn(kcache[qi, :, h, :], vbuf[vs, :, h, :], acc,
                             diag=True)
            obuf[os_, :, h, :] = acc / (qi + 1).astype(jnp.float32)

        @pl.when(mode == 1)
        def _marg():
            qf = _fmap(qcache[qi, :, h, :])
            num = jnp.dot(qf, kv_acc[h], preferred_element_type=jnp.float32)
            den = jnp.sum(qf * ks_acc[h], axis=1, keepdims=True)
            den = den + 1e-6 * (qi + 1).astype(jnp.float32)
            obuf[os_, :, h, :] = num / jnp.maximum(den, 1e-6)

        @pl.when(mode == 0)
        def _neg():
            obuf[os_, :, h, :] = jnp.zeros((BS, D), jnp.float32)

        @pl.when(h == H - 1)
        def _flush():
            pltpu.make_async_copy(obuf.at[os_],
                                  out_hbm.at[0, pl.ds(qi * BS, BS), :, :],
                                  osem.at[os_]).start()

            @pl.when(qi == n - 1)
            def _final_drain():
                pltpu.make_async_copy(
                    obuf.at[1 - os_],
                    out_hbm.at[0, pl.ds((qi - 1) * BS, BS), :, :],
                    osem.at[1 - os_]).wait()
                pltpu.make_async_copy(
                    obuf.at[os_],
                    out_hbm.at[0, pl.ds(qi * BS, BS), :, :],
                    osem.at[os_]).wait()


@jax.jit
def kernel(q, k, v):
    B, T, H, D = q.shape
    n = T // BS
    f32 = jnp.float32
    i32 = jnp.int32

    out = pl.pallas_call(
        _mega_kernel,
        grid=(2 * n, H),
        in_specs=[pl.BlockSpec(memory_space=pl.ANY)] * 3,
        out_specs=pl.BlockSpec(memory_space=pl.ANY),
        out_shape=jax.ShapeDtypeStruct((B, T, H, D), f32),
        scratch_shapes=[
            pltpu.VMEM((n, BS, H, D), f32),
            pltpu.VMEM((n, BS, H, D), f32),
            pltpu.VMEM((2, BS, H, D), f32),
            pltpu.VMEM((2, BS, H, D), f32),
            pltpu.VMEM((H, D, D), f32),
            pltpu.VMEM((H, 1, D), f32),
            pltpu.VMEM((n, H, D), f32),
            pltpu.VMEM((n, H, D), f32),
            pltpu.VMEM((4 * n, H), i32),
            pltpu.SMEM((4 * n, H), i32),
            pltpu.SMEM((1, 1), i32),
            pltpu.SMEM((1, 1), i32),
            pltpu.SemaphoreType.DMA((n,)),
            pltpu.SemaphoreType.DMA((n,)),
            pltpu.SemaphoreType.DMA((2,)),
            pltpu.SemaphoreType.DMA((2,)),
            pltpu.SemaphoreType.DMA,
            pltpu.SemaphoreType.DMA,
            pltpu.VMEM((BS, D), f32),
        ],
        compiler_params=pltpu.CompilerParams(
            dimension_semantics=("arbitrary", "arbitrary")),
    )(q, k, v)

    return out


# speedup vs baseline: 14.1349x; 1.0003x over previous
import math

import jax
import jax.numpy as jnp
from jax import lax
from jax.experimental import pallas as pl
from jax.experimental.pallas import tpu as pltpu

BS = 128
CRITICAL_T = 0.5
NEGLIGIBLE_T = 0.1
NEG = -1e9


def _fmap(x):
    return jnp.where(x > 0, x + 1.0, jnp.exp(x))


def _mega_kernel(q_hbm, k_hbm, v_hbm, out_hbm,
                 qcache, kcache, vbuf, obuf, kv_acc, ks_acc,
                 qrep_s, krep_s, tab_v, tab_s, slot_cur, slot_nxt,
                 qsem, ksem, vsem, osem, tsem, cvsem, cvbuf):
    n, _, H, D = qcache.shape
    gi = pl.program_id(0)
    h = pl.program_id(1)
    scale = 1.0 / math.sqrt(D)
    OFF_MODE, OFF_NKV, OFF_USE, OFF_NXT = 0, n, 2 * n, 3 * n

    def fetch_qk(i, sl):
        pltpu.make_async_copy(q_hbm.at[0, pl.ds(i * BS, BS), :, :],
                              qcache.at[sl], qsem.at[sl]).start()
        pltpu.make_async_copy(k_hbm.at[0, pl.ds(i * BS, BS), :, :],
                              kcache.at[sl], ksem.at[sl]).start()

    @pl.when((gi < n) & (h == 0))
    def _phase0():
        @pl.when(gi == 0)
        def _prime():
            for j in range(8):
                fetch_qk(j, j)

        pltpu.make_async_copy(q_hbm.at[0, pl.ds(gi * BS, BS), :, :],
                              qcache.at[gi], qsem.at[gi]).wait()
        pltpu.make_async_copy(k_hbm.at[0, pl.ds(gi * BS, BS), :, :],
                              kcache.at[gi], ksem.at[gi]).wait()
        qrep_s[gi] = jnp.mean(qcache[gi], axis=0)
        krep_s[gi] = jnp.mean(kcache[gi], axis=0)

        @pl.when(gi + 8 < n)
        def _ahead():
            fetch_qk(gi + 8, gi + 8)

        @pl.when(gi == n - 1)
        def _finalize():
            row = lax.broadcasted_iota(jnp.int32, (n, n), 0)
            col = lax.broadcasted_iota(jnp.int32, (n, n), 1)
            causal = col <= row
            masked = []
            for h2 in range(H):
                s = lax.dot_general(qrep_s[:, h2, :], krep_s[:, h2, :],
                                    (((1,), (1,)), ((), ())),
                                    preferred_element_type=jnp.float32) * scale
                masked.append(jnp.where(causal, s, NEG))
            gmin = jnp.min(masked[0])
            for h2 in range(1, H):
                gmin = jnp.minimum(gmin, jnp.min(masked[h2]))
            imps = []
            for h2 in range(H):
                s = masked[h2]
                m = jnp.max(s, axis=1, keepdims=True)
                e = jnp.exp(s - m)
                sm = e / jnp.sum(e, axis=1, keepdims=True)
                mu = jnp.mean(sm, axis=1, keepdims=True)
                var = jnp.sum((sm - mu) ** 2, axis=1, keepdims=True) / (n - 1)
                imps.append(var * jnp.maximum(m - gmin, 1e-6))
            gmax_i = jnp.max(imps[0])
            gmin_i = jnp.min(imps[0])
            for h2 in range(1, H):
                gmax_i = jnp.maximum(gmax_i, jnp.max(imps[h2]))
                gmin_i = jnp.minimum(gmin_i, jnp.min(imps[h2]))
            denom = gmax_i - gmin_i + 1e-6
            lane = lax.broadcasted_iota(jnp.int32, (n, H), 1)
            modes = jnp.zeros((n, H), jnp.int32)
            for h2 in range(H):
                inorm = (imps[h2] - gmin_i) / denom
                mcol = jnp.where(inorm >= CRITICAL_T, 2,
                                 jnp.where(inorm <= NEGLIGIBLE_T, 0,
                                           1)).astype(jnp.int32)
                modes = jnp.where(lane == h2, mcol, modes)
            marg = (modes == 1).astype(jnp.float32)
            upper = (col >= row).astype(jnp.float32)
            needkv = (lax.dot_general(upper, marg, (((1,), (0,)), ((), ())),
                                      preferred_element_type=jnp.float32)
                      > 0.5).astype(jnp.int32)
            used = (modes != 0) | (needkv == 1)
            anyuse = jnp.max(jnp.where(used, 1, 0), axis=1, keepdims=True)
            anyuse = jnp.broadcast_to(anyuse, (n, H)).astype(jnp.int32)
            nxt_rows = []
            cur = jnp.full((1, H), n, jnp.int32)
            for i2 in range(n - 1, -1, -1):
                cur = jnp.where(anyuse[i2:i2 + 1, :] == 1, i2, cur)
                nxt_rows.append(cur)
            nxt_rows.reverse()
            tab = jnp.concatenate(
                [modes, needkv, anyuse, jnp.concatenate(nxt_rows, axis=0)],
                axis=0)
            tab_v[...] = tab
            cp = pltpu.make_async_copy(tab_v, tab_s, tsem)
            cp.start()
            cp.wait()
            f0 = tab_s[OFF_NXT + 0, 0]
            slot_nxt[0, 0] = 0

            @pl.when(f0 < n)
            def _pf():
                pltpu.make_async_copy(v_hbm.at[0, pl.ds(f0 * BS, BS), :, :],
                                      vbuf.at[0], vsem.at[0]).start()

    qi = gi - n
    in_p1 = gi >= n

    @pl.when(in_p1)
    def _phase1():
        mode = tab_s[OFF_MODE + qi, h]
        nkv = tab_s[OFF_NKV + qi, h]
        au = tab_s[OFF_USE + qi, 0]
        os_ = lax.rem(qi, 2)

        @pl.when(qi == 0)
        def _reset():
            kv_acc[h] = jnp.zeros((D, D), jnp.float32)
            ks_acc[h] = jnp.zeros((1, D), jnp.float32)

        @pl.when(h == 0)
        def _head0():
            @pl.when(qi >= 2)
            def _drain():
                pltpu.make_async_copy(
                    obuf.at[os_],
                    out_hbm.at[0, pl.ds((qi - 2) * BS, BS), :, :],
                    osem.at[os_]).wait()

            @pl.when(au == 1)
            def _live():
                s = slot_nxt[0, 0]
                slot_cur[0, 0] = s
                slot_nxt[0, 0] = 1 - s
                pltpu.make_async_copy(
                    v_hbm.at[0, pl.ds(qi * BS, BS), :, :],
                    vbuf.at[s], vsem.at[s]).wait()

                @pl.when(qi < n - 1)
                def _pf_outer():
                    nx = tab_s[OFF_NXT + qi + 1, 0]

                    @pl.when(nx < n)
                    def _pf():
                        pltpu.make_async_copy(
                            v_hbm.at[0, pl.ds(nx * BS, BS), :, :],
                            vbuf.at[1 - s], vsem.at[1 - s]).start()

        vs = slot_cur[0, 0]

        @pl.when(nkv == 1)
        def _extend():
            kf = _fmap(kcache[qi, :, h, :])
            kv_acc[h] += lax.dot_general(kf, vbuf[vs, :, h, :],
                                         (((0,), (0,)), ((), ())),
                                         preferred_element_type=jnp.float32)
            ks_acc[h] += jnp.sum(kf, axis=0, keepdims=True)

        @pl.when(mode == 2)
        def _crit():
            qv = qcache[qi, :, h, :]

            def block_attn(kb, vb, acc, diag):
                s = lax.dot_general(qv, kb, (((1,), (1,)), ((), ())),
                                    preferred_element_type=jnp.float32) * scale
                if diag:
                    r = lax.broadcasted_iota(jnp.int32, (BS, BS), 0)
                    c = lax.broadcasted_iota(jnp.int32, (BS, BS), 1)
                    s = jnp.where(r >= c, s, NEG)
                m = jnp.max(s, axis=1, keepdims=True)
                e = jnp.exp(s - m)
                p = e / jnp.sum(e, axis=1, keepdims=True)
                return acc + lax.dot_general(p, vb, (((1,), (0,)), ((), ())),
                                             preferred_element_type=jnp.float32)

            def body(ki, acc):
                cv = pltpu.make_async_copy(
                    v_hbm.at[0, pl.ds(ki * BS, BS), h, :], cvbuf, cvsem)
                cv.start()
                cv.wait()
                return block_attn(kcache[ki, :, h, :], cvbuf[...], acc,
                                  diag=False)

            acc = lax.fori_loop(0, qi, body, jnp.zeros((BS, D), jnp.float32))
            acc = block_attn(kcache[qi, :, h, :], vbuf[vs, :, h, :], acc,
                             diag=True)
            obuf[os_, :, h, :] = acc / (qi + 1).astype(jnp.float32)

        @pl.when(mode == 1)
        def _marg():
            qf = _fmap(qcache[qi, :, h, :])
            num = jnp.dot(qf, kv_acc[h], preferred_element_type=jnp.float32)
            den = jnp.sum(qf * ks_acc[h], axis=1, keepdims=True)
            den = den + 1e-6 * (qi + 1).astype(jnp.float32)
            obuf[os_, :, h, :] = num / jnp.maximum(den, 1e-6)

        @pl.when(mode == 0)
        def _neg():
            obuf[os_, :, h, :] = jnp.zeros((BS, D), jnp.float32)

        @pl.when(h == H - 1)
        def _flush():
            pltpu.make_async_copy(obuf.at[os_],
                                  out_hbm.at[0, pl.ds(qi * BS, BS), :, :],
                                  osem.at[os_]).start()

            @pl.when(qi == n - 1)
            def _final_drain():
                pltpu.make_async_copy(
                    obuf.at[1 - os_],
                    out_hbm.at[0, pl.ds((qi - 1) * BS, BS), :, :],
                    osem.at[1 - os_]).wait()
                pltpu.make_async_copy(
                    obuf.at[os_],
                    out_hbm.at[0, pl.ds(qi * BS, BS), :, :],
                    osem.at[os_]).wait()


@jax.jit
def kernel(q, k, v):
    B, T, H, D = q.shape
    n = T // BS
    f32 = jnp.float32
    i32 = jnp.int32

    out = pl.pallas_call(
        _mega_kernel,
        grid=(2 * n, H),
        in_specs=[pl.BlockSpec(memory_space=pl.ANY)] * 3,
        out_specs=pl.BlockSpec(memory_space=pl.ANY),
        out_shape=jax.ShapeDtypeStruct((B, T, H, D), f32),
        scratch_shapes=[
            pltpu.VMEM((n, BS, H, D), f32),
            pltpu.VMEM((n, BS, H, D), f32),
            pltpu.VMEM((2, BS, H, D), f32),
            pltpu.VMEM((2, BS, H, D), f32),
            pltpu.VMEM((H, D, D), f32),
            pltpu.VMEM((H, 1, D), f32),
            pltpu.VMEM((n, H, D), f32),
            pltpu.VMEM((n, H, D), f32),
            pltpu.VMEM((4 * n, H), i32),
            pltpu.SMEM((4 * n, H), i32),
            pltpu.SMEM((1, 1), i32),
            pltpu.SMEM((1, 1), i32),
            pltpu.SemaphoreType.DMA((n,)),
            pltpu.SemaphoreType.DMA((n,)),
            pltpu.SemaphoreType.DMA((2,)),
            pltpu.SemaphoreType.DMA((2,)),
            pltpu.SemaphoreType.DMA,
            pltpu.SemaphoreType.DMA,
            pltpu.VMEM((BS, D), f32),
        ],
        compiler_params=pltpu.CompilerParams(
            dimension_semantics=("arbitrary", "arbitrary")),
    )(q, k, v)

    return out
